# Initial kernel scaffold; baseline (speedup 1.0000x reference)
import jax
import jax.numpy as jnp
from jax.experimental import pallas as pl


def kernel(lat, codebook, leak_factor):
    raise NotImplementedError("write your pallas kernel here")


# speedup vs baseline: 3.7453x; 3.7453x over previous
import functools

import jax
import jax.numpy as jnp
from jax.experimental import pallas as pl
from jax.experimental.pallas import tpu as pltpu

LAT_SIZE = 4096
N_FILTER = 32
NCENTS = 1024
BATCH = 4096
BETA = 0.25

TB = 32


CHUNK = 512


def _vq_kernel(lat_ref, xr_ref, cb_ref, lf_ref, out_ref, acc_ref):
    i = pl.program_id(0)
    x = lat_ref[...]
    out_ref[...] = x * lf_ref[0, 0]

    cbt = cb_ref[...].astype(jnp.bfloat16)
    c2 = jnp.sum(cb_ref[...] * cb_ref[...], axis=0)
    rb = TB * (LAT_SIZE // N_FILTER)

    def body(j, acc):
        chunk = xr_ref[pl.ds(j * CHUNK, CHUNK), :]
        xc = jax.lax.dot_general(
            chunk.astype(jnp.bfloat16), cbt,
            (((1,), (0,)), ((), ())),
            preferred_element_type=jnp.float32)
        m = jnp.min(c2[None, :] - 2.0 * xc, axis=1)
        return acc + jnp.sum(m)

    s = jax.lax.fori_loop(0, rb // CHUNK, body, 0.0)
    partial = (jnp.sum(x * x) + s).reshape(1, 1)

    @pl.when(i == 0)
    def _():
        acc_ref[...] = jnp.zeros_like(acc_ref)
    acc_ref[...] += partial


@functools.partial(jax.jit, static_argnames=())
def kernel(lat, codebook, leak_factor):
    lf = jnp.clip(leak_factor, 0.001, 1000.0).reshape(1, 1)
    xr = lat.reshape(BATCH * (LAT_SIZE // N_FILTER), N_FILTER)
    grid = (BATCH // TB,)
    rb = TB * (LAT_SIZE // N_FILTER)
    out, acc = pl.pallas_call(
        _vq_kernel,
        grid=grid,
        in_specs=[
            pl.BlockSpec((TB, LAT_SIZE), lambda i: (i, 0)),
            pl.BlockSpec((rb, N_FILTER), lambda i: (i, 0)),
            pl.BlockSpec((N_FILTER, NCENTS), lambda i: (0, 0)),
            pl.BlockSpec(memory_space=pltpu.SMEM),
        ],
        out_specs=[
            pl.BlockSpec((TB, LAT_SIZE), lambda i: (i, 0)),
            pl.BlockSpec((1, 1), lambda i: (0, 0)),
        ],
        out_shape=[
            jax.ShapeDtypeStruct((BATCH, LAT_SIZE), jnp.float32),
            jax.ShapeDtypeStruct((1, 1), jnp.float32),
        ],
    )(lat, xr, codebook.T, lf)
    numel = BATCH * LAT_SIZE
    cent_loss = acc[0, 0] * ((1.0 + BETA) / numel)
    return out, cent_loss


# speedup vs baseline: 3.8721x; 1.0339x over previous
import functools

import jax
import jax.numpy as jnp
from jax.experimental import pallas as pl
from jax.experimental.pallas import tpu as pltpu

LAT_SIZE = 4096
N_FILTER = 32
NCENTS = 1024
BATCH = 4096
BETA = 0.25

TB = 64
CHUNK = 1024
KAUG = 40


def _vq_kernel(lat_ref, xr_ref, cb_ref, lf_ref, out_ref, acc_ref):
    i = pl.program_id(0)
    x = lat_ref[...]
    out_ref[...] = x * lf_ref[0, 0]

    cbt = cb_ref[...].astype(jnp.bfloat16)
    rb = TB * (LAT_SIZE // N_FILTER)

    def body(j, acc):
        chunk = xr_ref[pl.ds(j * CHUNK, CHUNK), :]
        d = jax.lax.dot_general(
            chunk.astype(jnp.bfloat16), cbt,
            (((1,), (0,)), ((), ())),
            preferred_element_type=jnp.float32)
        return acc + jnp.sum(jnp.min(d, axis=1))

    s = jax.lax.fori_loop(0, rb // CHUNK, body, 0.0)
    partial = (jnp.sum(x * x) + s).reshape(1, 1)

    @pl.when(i == 0)
    def _():
        acc_ref[...] = jnp.zeros_like(acc_ref)
    acc_ref[...] += partial


@functools.partial(jax.jit, static_argnames=())
def kernel(lat, codebook, leak_factor):
    lf = jnp.clip(leak_factor, 0.001, 1000.0).reshape(1, 1)
    nrows = BATCH * (LAT_SIZE // N_FILTER)
    xr = lat.reshape(nrows, N_FILTER)
    pad = jnp.concatenate(
        [jnp.ones((nrows, 1), jnp.float32),
         jnp.zeros((nrows, KAUG - N_FILTER - 1), jnp.float32)], axis=1)
    xr_aug = jnp.concatenate([xr, pad], axis=1)
    c2 = jnp.sum(codebook * codebook, axis=1)
    cbt_aug = jnp.concatenate(
        [-2.0 * codebook.T, c2[None, :],
         jnp.zeros((KAUG - N_FILTER - 1, NCENTS), jnp.float32)], axis=0)

    grid = (BATCH // TB,)
    rb = TB * (LAT_SIZE // N_FILTER)
    out, acc = pl.pallas_call(
        _vq_kernel,
        grid=grid,
        in_specs=[
            pl.BlockSpec((TB, LAT_SIZE), lambda i: (i, 0)),
            pl.BlockSpec((rb, KAUG), lambda i: (i, 0)),
            pl.BlockSpec((KAUG, NCENTS), lambda i: (0, 0)),
            pl.BlockSpec(memory_space=pltpu.SMEM),
        ],
        out_specs=[
            pl.BlockSpec((TB, LAT_SIZE), lambda i: (i, 0)),
            pl.BlockSpec((1, 1), lambda i: (0, 0)),
        ],
        out_shape=[
            jax.ShapeDtypeStruct((BATCH, LAT_SIZE), jnp.float32),
            jax.ShapeDtypeStruct((1, 1), jnp.float32),
        ],
    )(lat, xr_aug, cbt_aug, lf)
    numel = BATCH * LAT_SIZE
    cent_loss = acc[0, 0] * ((1.0 + BETA) / numel)
    return out, cent_loss


# speedup vs baseline: 4.9934x; 1.2896x over previous
import functools

import jax
import jax.numpy as jnp
from jax.experimental import pallas as pl
from jax.experimental.pallas import tpu as pltpu

LAT_SIZE = 4096
N_FILTER = 32
NCENTS = 1024
BATCH = 4096
BETA = 0.25

TB = 64
CHUNK = 1024
KAUG = 40


def _vq_kernel(lat_ref, xr_ref, cb_ref, lf_ref, out_ref, acc_ref):
    i = pl.program_id(0)
    x = lat_ref[...]
    out_ref[...] = x * lf_ref[0, 0]

    cbt = cb_ref[...]
    rb = TB * (LAT_SIZE // N_FILTER)

    s = 0.0
    for j in range(rb // CHUNK):
        chunk = xr_ref[pl.ds(j * CHUNK, CHUNK), :]
        d = jax.lax.dot_general(
            chunk, cbt,
            (((1,), (0,)), ((), ())),
            preferred_element_type=jnp.float32)
        s = s + jnp.sum(jnp.min(d, axis=1))

    partial = (jnp.sum(x * x) + s).reshape(1, 1)

    @pl.when(i == 0)
    def _():
        acc_ref[...] = jnp.zeros_like(acc_ref)
    acc_ref[...] += partial


@functools.partial(jax.jit, static_argnames=())
def kernel(lat, codebook, leak_factor):
    lf = jnp.clip(leak_factor, 0.001, 1000.0).reshape(1, 1)
    nrows = BATCH * (LAT_SIZE // N_FILTER)
    xr = lat.reshape(nrows, N_FILTER)
    pad = jnp.concatenate(
        [jnp.ones((nrows, 1), jnp.float32),
         jnp.zeros((nrows, KAUG - N_FILTER - 1), jnp.float32)], axis=1)
    xr_aug = jnp.concatenate([xr, pad], axis=1)
    c2 = jnp.sum(codebook * codebook, axis=1)
    cbt_aug = jnp.concatenate(
        [-2.0 * codebook.T, c2[None, :],
         jnp.zeros((KAUG - N_FILTER - 1, NCENTS), jnp.float32)], axis=0)

    grid = (BATCH // TB,)
    rb = TB * (LAT_SIZE // N_FILTER)
    out, acc = pl.pallas_call(
        _vq_kernel,
        grid=grid,
        in_specs=[
            pl.BlockSpec((TB, LAT_SIZE), lambda i: (i, 0)),
            pl.BlockSpec((rb, KAUG), lambda i: (i, 0)),
            pl.BlockSpec((KAUG, NCENTS), lambda i: (0, 0)),
            pl.BlockSpec(memory_space=pltpu.SMEM),
        ],
        out_specs=[
            pl.BlockSpec((TB, LAT_SIZE), lambda i: (i, 0)),
            pl.BlockSpec((1, 1), lambda i: (0, 0)),
        ],
        out_shape=[
            jax.ShapeDtypeStruct((BATCH, LAT_SIZE), jnp.float32),
            jax.ShapeDtypeStruct((1, 1), jnp.float32),
        ],
    )(lat, xr_aug, cbt_aug, lf)
    numel = BATCH * LAT_SIZE
    cent_loss = acc[0, 0] * ((1.0 + BETA) / numel)
    return out, cent_loss


# speedup vs baseline: 5.0843x; 1.0182x over previous
import functools

import jax
import jax.numpy as jnp
from jax.experimental import pallas as pl
from jax.experimental.pallas import tpu as pltpu

LAT_SIZE = 4096
N_FILTER = 32
NCENTS = 1024
BATCH = 4096
BETA = 0.25

TB = 128
CHUNK = 1024
KAUG = 40


def _vq_kernel(lat_ref, xr_ref, cb_ref, lf_ref, out_ref, acc_ref):
    i = pl.program_id(0)
    x = lat_ref[...]
    out_ref[...] = x * lf_ref[0, 0]

    cbt = cb_ref[...]
    rb = TB * (LAT_SIZE // N_FILTER)

    s = 0.0
    for j in range(rb // CHUNK):
        chunk = xr_ref[pl.ds(j * CHUNK, CHUNK), :]
        d = jax.lax.dot_general(
            chunk, cbt,
            (((1,), (0,)), ((), ())),
            preferred_element_type=jnp.float32)
        s = s + jnp.sum(jnp.min(d, axis=1))

    partial = (jnp.sum(x * x) + s).reshape(1, 1)

    @pl.when(i == 0)
    def _():
        acc_ref[...] = jnp.zeros_like(acc_ref)
    acc_ref[...] += partial


@functools.partial(jax.jit, static_argnames=())
def kernel(lat, codebook, leak_factor):
    lf = jnp.clip(leak_factor, 0.001, 1000.0).reshape(1, 1)
    nrows = BATCH * (LAT_SIZE // N_FILTER)
    xr = lat.reshape(nrows, N_FILTER)
    pad = jnp.concatenate(
        [jnp.ones((nrows, 1), jnp.float32),
         jnp.zeros((nrows, KAUG - N_FILTER - 1), jnp.float32)], axis=1)
    xr_aug = jnp.concatenate([xr, pad], axis=1)
    c2 = jnp.sum(codebook * codebook, axis=1)
    cbt_aug = jnp.concatenate(
        [-2.0 * codebook.T, c2[None, :],
         jnp.zeros((KAUG - N_FILTER - 1, NCENTS), jnp.float32)], axis=0)

    grid = (BATCH // TB,)
    rb = TB * (LAT_SIZE // N_FILTER)
    out, acc = pl.pallas_call(
        _vq_kernel,
        grid=grid,
        in_specs=[
            pl.BlockSpec((TB, LAT_SIZE), lambda i: (i, 0)),
            pl.BlockSpec((rb, KAUG), lambda i: (i, 0)),
            pl.BlockSpec((KAUG, NCENTS), lambda i: (0, 0)),
            pl.BlockSpec(memory_space=pltpu.SMEM),
        ],
        out_specs=[
            pl.BlockSpec((TB, LAT_SIZE), lambda i: (i, 0)),
            pl.BlockSpec((1, 1), lambda i: (0, 0)),
        ],
        out_shape=[
            jax.ShapeDtypeStruct((BATCH, LAT_SIZE), jnp.float32),
            jax.ShapeDtypeStruct((1, 1), jnp.float32),
        ],
    )(lat, xr_aug, cbt_aug, lf)
    numel = BATCH * LAT_SIZE
    cent_loss = acc[0, 0] * ((1.0 + BETA) / numel)
    return out, cent_loss


# speedup vs baseline: 6.8750x; 1.3522x over previous
import functools

import jax
import jax.numpy as jnp
from jax.experimental import pallas as pl
from jax.experimental.pallas import tpu as pltpu

LAT_SIZE = 4096
N_FILTER = 32
NCENTS = 1024
BATCH = 4096
BETA = 0.25

TB = 128
CHUNK = 1024


def _vq_kernel(lat_ref, xr_ref, cb_ref, c2_ref, lf_ref, out_ref, acc_ref):
    i = pl.program_id(0)
    x = lat_ref[...]
    out_ref[...] = x * lf_ref[0, 0]

    cbt = cb_ref[...]
    c2 = c2_ref[...]
    rb = TB * (LAT_SIZE // N_FILTER)

    s = 0.0
    for j in range(rb // CHUNK):
        chunk = xr_ref[pl.ds(j * CHUNK, CHUNK), :]
        d = jax.lax.dot_general(
            chunk, cbt,
            (((1,), (0,)), ((), ())),
            preferred_element_type=jnp.float32)
        s = s + jnp.sum(jnp.min(d + c2, axis=1))

    partial = (jnp.sum(x * x) + s).reshape(1, 1)

    @pl.when(i == 0)
    def _():
        acc_ref[...] = jnp.zeros_like(acc_ref)
    acc_ref[...] += partial


@functools.partial(jax.jit, static_argnames=())
def kernel(lat, codebook, leak_factor):
    lf = jnp.clip(leak_factor, 0.001, 1000.0).reshape(1, 1)
    nrows = BATCH * (LAT_SIZE // N_FILTER)
    xr = lat.reshape(nrows, N_FILTER)
    c2 = jnp.sum(codebook * codebook, axis=1).reshape(1, NCENTS)
    cbt = -2.0 * codebook.T

    grid = (BATCH // TB,)
    rb = TB * (LAT_SIZE // N_FILTER)
    out, acc = pl.pallas_call(
        _vq_kernel,
        grid=grid,
        in_specs=[
            pl.BlockSpec((TB, LAT_SIZE), lambda i: (i, 0)),
            pl.BlockSpec((rb, N_FILTER), lambda i: (i, 0)),
            pl.BlockSpec((N_FILTER, NCENTS), lambda i: (0, 0)),
            pl.BlockSpec((1, NCENTS), lambda i: (0, 0)),
            pl.BlockSpec(memory_space=pltpu.SMEM),
        ],
        out_specs=[
            pl.BlockSpec((TB, LAT_SIZE), lambda i: (i, 0)),
            pl.BlockSpec((1, 1), lambda i: (0, 0)),
        ],
        out_shape=[
            jax.ShapeDtypeStruct((BATCH, LAT_SIZE), jnp.float32),
            jax.ShapeDtypeStruct((1, 1), jnp.float32),
        ],
    )(lat, xr, cbt, c2, lf)
    numel = BATCH * LAT_SIZE
    cent_loss = acc[0, 0] * ((1.0 + BETA) / numel)
    return out, cent_loss


# speedup vs baseline: 8.6809x; 1.2627x over previous
import functools

import jax
import jax.numpy as jnp
from jax.experimental import pallas as pl
from jax.experimental.pallas import tpu as pltpu

LAT_SIZE = 4096
N_FILTER = 32
NCENTS = 1024
BATCH = 4096
BETA = 0.25

TB = 128
CHUNK = 1024


def _vq_kernel(lat_ref, xr_ref, cb_ref, c2_ref, lf_ref, out_ref, acc_ref):
    i = pl.program_id(0)
    x = lat_ref[...]
    out_ref[...] = x * lf_ref[0, 0]

    cbt = cb_ref[...]
    c2 = c2_ref[...]
    rb = TB * (LAT_SIZE // N_FILTER)

    s = 0.0
    for j in range(rb // CHUNK):
        chunk = xr_ref[pl.ds(j * CHUNK, CHUNK), :]
        d = jax.lax.dot_general(
            chunk, cbt,
            (((1,), (0,)), ((), ())),
            preferred_element_type=jnp.float32)
        s = s + jnp.sum(jnp.min(d + c2, axis=1))

    partial = (jnp.sum(x * x) + s).reshape(1, 1)

    @pl.when(i == 0)
    def _():
        acc_ref[...] = jnp.zeros_like(acc_ref)
    acc_ref[...] += partial


@functools.partial(jax.jit, static_argnames=())
def kernel(lat, codebook, leak_factor):
    lf = jnp.clip(leak_factor, 0.001, 1000.0).reshape(1, 1)
    nrows = BATCH * (LAT_SIZE // N_FILTER)
    xr = lat.reshape(nrows, N_FILTER).astype(jnp.float8_e4m3fn)
    c2 = jnp.sum(codebook * codebook, axis=1).reshape(1, NCENTS)
    cbt = (-2.0 * codebook.T).astype(jnp.float8_e4m3fn)

    grid = (BATCH // TB,)
    rb = TB * (LAT_SIZE // N_FILTER)
    out, acc = pl.pallas_call(
        _vq_kernel,
        grid=grid,
        in_specs=[
            pl.BlockSpec((TB, LAT_SIZE), lambda i: (i, 0)),
            pl.BlockSpec((rb, N_FILTER), lambda i: (i, 0)),
            pl.BlockSpec((N_FILTER, NCENTS), lambda i: (0, 0)),
            pl.BlockSpec((1, NCENTS), lambda i: (0, 0)),
            pl.BlockSpec(memory_space=pltpu.SMEM),
        ],
        out_specs=[
            pl.BlockSpec((TB, LAT_SIZE), lambda i: (i, 0)),
            pl.BlockSpec((1, 1), lambda i: (0, 0)),
        ],
        out_shape=[
            jax.ShapeDtypeStruct((BATCH, LAT_SIZE), jnp.float32),
            jax.ShapeDtypeStruct((1, 1), jnp.float32),
        ],
    )(lat, xr, cbt, c2, lf)
    numel = BATCH * LAT_SIZE
    cent_loss = acc[0, 0] * ((1.0 + BETA) / numel)
    return out, cent_loss


# speedup vs baseline: 12.1349x; 1.3979x over previous
import functools

import jax
import jax.numpy as jnp
from jax.experimental import pallas as pl
from jax.experimental.pallas import tpu as pltpu

LAT_SIZE = 4096
N_FILTER = 32
NCENTS = 1024
BATCH = 4096
BETA = 0.25

TB = 128
PACK = 8
SL = PACK * N_FILTER
NSL = LAT_SIZE // SL


def _vq_kernel(lat_ref, cb_ref, c2_ref, lf_ref, out_ref, acc_ref):
    i = pl.program_id(0)
    x = lat_ref[...]
    out_ref[...] = x * lf_ref[0, 0]

    cbd = cb_ref[...]
    c2 = c2_ref[...]
    x8 = x.astype(jnp.float8_e4m3fn)

    s = 0.0
    for j in range(NSL):
        sl = jax.lax.slice(x8, (0, j * SL), (TB, (j + 1) * SL))
        d = jax.lax.dot_general(
            sl, cbd,
            (((1,), (0,)), ((), ())),
            preferred_element_type=jnp.float32)
        for k in range(PACK):
            dk = jax.lax.slice(d, (0, k * NCENTS), (TB, (k + 1) * NCENTS))
            s = s + jnp.sum(jnp.min(dk + c2, axis=1))

    partial = (jnp.sum(x * x) + s).reshape(1, 1)

    @pl.when(i == 0)
    def _():
        acc_ref[...] = jnp.zeros_like(acc_ref)
    acc_ref[...] += partial


@functools.partial(jax.jit, static_argnames=())
def kernel(lat, codebook, leak_factor):
    lf = jnp.clip(leak_factor, 0.001, 1000.0).reshape(1, 1)
    c2 = jnp.sum(codebook * codebook, axis=1).reshape(1, NCENTS)
    cbt = -2.0 * codebook.T
    eye = jnp.eye(PACK, dtype=jnp.float32)
    cbd = jnp.einsum('pq,fk->pfqk', eye, cbt).reshape(
        SL, PACK * NCENTS).astype(jnp.float8_e4m3fn)

    grid = (BATCH // TB,)
    out, acc = pl.pallas_call(
        _vq_kernel,
        grid=grid,
        in_specs=[
            pl.BlockSpec((TB, LAT_SIZE), lambda i: (i, 0)),
            pl.BlockSpec((SL, PACK * NCENTS), lambda i: (0, 0)),
            pl.BlockSpec((1, NCENTS), lambda i: (0, 0)),
            pl.BlockSpec(memory_space=pltpu.SMEM),
        ],
        out_specs=[
            pl.BlockSpec((TB, LAT_SIZE), lambda i: (i, 0)),
            pl.BlockSpec((1, 1), lambda i: (0, 0)),
        ],
        out_shape=[
            jax.ShapeDtypeStruct((BATCH, LAT_SIZE), jnp.float32),
            jax.ShapeDtypeStruct((1, 1), jnp.float32),
        ],
    )(lat, cbd, c2, lf)
    numel = BATCH * LAT_SIZE
    cent_loss = acc[0, 0] * ((1.0 + BETA) / numel)
    return out, cent_loss


# speedup vs baseline: 14.4538x; 1.1911x over previous
import functools

import jax
import jax.numpy as jnp
from jax.experimental import pallas as pl
from jax.experimental.pallas import tpu as pltpu

LAT_SIZE = 4096
N_FILTER = 32
NCENTS = 1024
BATCH = 4096
BETA = 0.25

TB = 256
PACK = 8
SL = PACK * N_FILTER
NSL = LAT_SIZE // SL


def _vq_kernel(lat_ref, cb_ref, c2_ref, lf_ref, out_ref, acc_ref):
    i = pl.program_id(0)
    x = lat_ref[...]
    out_ref[...] = x * lf_ref[0, 0]

    cbd = cb_ref[...]
    c2 = c2_ref[...]
    x8 = x.astype(jnp.float8_e4m3fn)

    s = 0.0
    for j in range(NSL):
        sl = jax.lax.slice(x8, (0, j * SL), (TB, (j + 1) * SL))
        d = jax.lax.dot_general(
            sl, cbd,
            (((1,), (0,)), ((), ())),
            preferred_element_type=jnp.float32)
        for k in range(PACK):
            dk = jax.lax.slice(d, (0, k * NCENTS), (TB, (k + 1) * NCENTS))
            s = s + jnp.sum(jnp.min(dk + c2, axis=1))

    partial = (jnp.sum(x * x) + s).reshape(1, 1)

    @pl.when(i == 0)
    def _():
        acc_ref[...] = jnp.zeros_like(acc_ref)
    acc_ref[...] += partial


@functools.partial(jax.jit, static_argnames=())
def kernel(lat, codebook, leak_factor):
    lf = jnp.clip(leak_factor, 0.001, 1000.0).reshape(1, 1)
    c2 = jnp.sum(codebook * codebook, axis=1).reshape(1, NCENTS)
    cbt = -2.0 * codebook.T
    eye = jnp.eye(PACK, dtype=jnp.float32)
    cbd = jnp.einsum('pq,fk->pfqk', eye, cbt).reshape(
        SL, PACK * NCENTS).astype(jnp.float8_e4m3fn)

    grid = (BATCH // TB,)
    out, acc = pl.pallas_call(
        _vq_kernel,
        grid=grid,
        in_specs=[
            pl.BlockSpec((TB, LAT_SIZE), lambda i: (i, 0)),
            pl.BlockSpec((SL, PACK * NCENTS), lambda i: (0, 0)),
            pl.BlockSpec((1, NCENTS), lambda i: (0, 0)),
            pl.BlockSpec(memory_space=pltpu.SMEM),
        ],
        out_specs=[
            pl.BlockSpec((TB, LAT_SIZE), lambda i: (i, 0)),
            pl.BlockSpec((1, 1), lambda i: (0, 0)),
        ],
        out_shape=[
            jax.ShapeDtypeStruct((BATCH, LAT_SIZE), jnp.float32),
            jax.ShapeDtypeStruct((1, 1), jnp.float32),
        ],
    )(lat, cbd, c2, lf)
    numel = BATCH * LAT_SIZE
    cent_loss = acc[0, 0] * ((1.0 + BETA) / numel)
    return out, cent_loss


# speedup vs baseline: 16.8098x; 1.1630x over previous
import functools

import jax
import jax.numpy as jnp
from jax.experimental import pallas as pl
from jax.experimental.pallas import tpu as pltpu

LAT_SIZE = 4096
N_FILTER = 32
NCENTS = 1024
BATCH = 4096
BETA = 0.25

TB = 256
PACK = 8
SL = PACK * N_FILTER
NSL = LAT_SIZE // SL


def _vq_kernel(lat_ref, cb_ref, c2_ref, lf_ref, out_ref, acc_ref):
    i = pl.program_id(0)
    x = lat_ref[...]
    out_ref[...] = x * lf_ref[0, 0]

    cbd = cb_ref[...]
    c2 = c2_ref[...].astype(jnp.bfloat16)
    x8 = x.astype(jnp.float8_e4m3fn)

    s = 0.0
    for k in range(PACK):
        cbk = cbd[:, k * NCENTS:(k + 1) * NCENTS]
        for j in range(NSL):
            sl = jax.lax.slice(x8, (0, j * SL), (TB, (j + 1) * SL))
            dk = jax.lax.dot_general(
                sl, cbk,
                (((1,), (0,)), ((), ())),
                preferred_element_type=jnp.float32)
            m = jnp.min(dk.astype(jnp.bfloat16) + c2, axis=1)
            s = s + jnp.sum(m.astype(jnp.float32))

    partial = (jnp.sum(x * x) + s).reshape(1, 1)

    @pl.when(i == 0)
    def _():
        acc_ref[...] = jnp.zeros_like(acc_ref)
    acc_ref[...] += partial


@functools.partial(jax.jit, static_argnames=())
def kernel(lat, codebook, leak_factor):
    lf = jnp.clip(leak_factor, 0.001, 1000.0).reshape(1, 1)
    c2 = jnp.sum(codebook * codebook, axis=1).reshape(1, NCENTS)
    cbt = -2.0 * codebook.T
    eye = jnp.eye(PACK, dtype=jnp.float32)
    cbd = jnp.einsum('pq,fk->pfqk', eye, cbt).reshape(
        SL, PACK * NCENTS).astype(jnp.float8_e4m3fn)

    grid = (BATCH // TB,)
    out, acc = pl.pallas_call(
        _vq_kernel,
        grid=grid,
        in_specs=[
            pl.BlockSpec((TB, LAT_SIZE), lambda i: (i, 0)),
            pl.BlockSpec((SL, PACK * NCENTS), lambda i: (0, 0)),
            pl.BlockSpec((1, NCENTS), lambda i: (0, 0)),
            pl.BlockSpec(memory_space=pltpu.SMEM),
        ],
        out_specs=[
            pl.BlockSpec((TB, LAT_SIZE), lambda i: (i, 0)),
            pl.BlockSpec((1, 1), lambda i: (0, 0)),
        ],
        out_shape=[
            jax.ShapeDtypeStruct((BATCH, LAT_SIZE), jnp.float32),
            jax.ShapeDtypeStruct((1, 1), jnp.float32),
        ],
    )(lat, cbd, c2, lf)
    numel = BATCH * LAT_SIZE
    cent_loss = acc[0, 0] * ((1.0 + BETA) / numel)
    return out, cent_loss
